# baseline (speedup 1.0000x reference)
import functools

import jax
import jax.numpy as jnp
from jax import lax
from jax.experimental import pallas as pl
from jax.experimental.pallas import tpu as pltpu
from jax.experimental.pallas import tpu_sc as plsc

_PATTERNS = [
    [(i * 7) % 100, (i * 7 + 13) % 100, (i * 7 + 29) % 100, (i * 7 + 53) % 100]
    for i in range(26)
]
_B = 16384
_W = 25


@functools.lru_cache(maxsize=None)
def _build_sc_match():
    info = plsc.get_sparse_core_info()
    nc, ns, lanes = info.num_cores, info.num_subcores, info.num_lanes
    nw = nc * ns
    rows_per_w = _B // nw
    chunks = rows_per_w // lanes
    mesh = plsc.VectorSubcoreMesh(core_axis_name="c", subcore_axis_name="s")

    @functools.partial(
        pl.kernel,
        mesh=mesh,
        out_type=jax.ShapeDtypeStruct((_B,), jnp.int32),
        scratch_types=[
            pltpu.VMEM((rows_per_w * _W,), jnp.int32),
            pltpu.VMEM((rows_per_w,), jnp.int32),
        ],
        compiler_params=pltpu.CompilerParams(needs_layout_passes=False),
    )
    def sc_match(words_hbm, out_hbm, wbuf, obuf):
        wid = lax.axis_index("s") * nc + lax.axis_index("c")
        row0 = wid * rows_per_w
        pltpu.sync_copy(words_hbm.at[pl.ds(row0 * _W, rows_per_w * _W)], wbuf)

        lane_off = lax.iota(jnp.int32, lanes) * _W

        def body(c, _):
            base = lane_off + c * (lanes * _W)
            words = [plsc.load_gather(wbuf, [base + w]) for w in range(_W)]
            acc = None
            for pat in _PATTERNS:
                m = None
                for col in pat:
                    v = words[col // 4]
                    sh = 8 * (col % 4)
                    if sh:
                        v = lax.shift_right_logical(v, sh)
                    m = v if m is None else (m & v)
                acc = m if acc is None else (acc | m)
            obuf[pl.ds(c * lanes, lanes)] = acc & 1
            return _

        lax.fori_loop(0, chunks, body, None)
        pltpu.sync_copy(obuf, out_hbm.at[pl.ds(row0, rows_per_w)])

    return sc_match


def kernel(inputs):
    words = lax.bitcast_convert_type(
        inputs.astype(jnp.uint8).reshape(_B, _W, 4), jnp.int32
    ).reshape(_B * _W)
    out = _build_sc_match()(words)
    return out != 0


# speedup vs baseline: 1.7085x; 1.7085x over previous
import functools

import jax
import jax.numpy as jnp
from jax import lax
from jax.experimental import pallas as pl
from jax.experimental.pallas import tpu as pltpu
from jax.experimental.pallas import tpu_sc as plsc

_PATTERNS = [
    [(i * 7) % 100, (i * 7 + 13) % 100, (i * 7 + 29) % 100, (i * 7 + 53) % 100]
    for i in range(26)
]
_B = 16384
_F = 100
_R = _B // 4


@functools.lru_cache(maxsize=None)
def _build_sc_match():
    info = plsc.get_sparse_core_info()
    nc, ns, lanes = info.num_cores, info.num_subcores, info.num_lanes
    nw = nc * ns
    wrows_per_w = _R // nw
    chunks = wrows_per_w // lanes
    mesh = plsc.VectorSubcoreMesh(core_axis_name="c", subcore_axis_name="s")

    @functools.partial(
        pl.kernel,
        mesh=mesh,
        out_type=jax.ShapeDtypeStruct((_R,), jnp.int32),
        scratch_types=[
            pltpu.VMEM((wrows_per_w, _F), jnp.int32),
            pltpu.VMEM((wrows_per_w,), jnp.int32),
        ],
        compiler_params=pltpu.CompilerParams(needs_layout_passes=False),
    )
    def sc_match(in_hbm, out_hbm, wbuf, obuf):
        wid = lax.axis_index("s") * nc + lax.axis_index("c")
        wrow0 = wid * wrows_per_w
        words_hbm = in_hbm.bitcast(jnp.int32)
        pltpu.sync_copy(words_hbm.at[pl.ds(wrow0, wrows_per_w), :], wbuf)

        lane = lax.iota(jnp.int32, lanes)

        def body(c, carry):
            ridx = lane + c * lanes
            acc = None
            for pat in _PATTERNS:
                m = None
                for col in pat:
                    v = plsc.load_gather(wbuf, [ridx, jnp.full((lanes,), col, jnp.int32)])
                    m = v if m is None else (m & v)
                acc = m if acc is None else (acc | m)
            obuf[pl.ds(c * lanes, lanes)] = acc
            return carry

        lax.fori_loop(0, chunks, body, None)
        pltpu.sync_copy(obuf, out_hbm.at[pl.ds(wrow0, wrows_per_w)])

    return sc_match


def kernel(inputs):
    out_words = _build_sc_match()(inputs.astype(jnp.uint8))
    return lax.bitcast_convert_type(out_words, jnp.uint8).reshape(_B) != 0
